# Initial kernel scaffold; baseline (speedup 1.0000x reference)
import jax
import jax.numpy as jnp
from jax.experimental import pallas as pl


def kernel(cell, x, z, num_atoms, enc_emb, enc_edge_w, enc_upd_w, W_energy, dec_in_w, dec_edge_w, dec_upd_w, dec_force_w, dec_stress_w):
    raise NotImplementedError("write your pallas kernel here")


# speedup vs baseline: 24.1905x; 24.1905x over previous
import jax
import jax.numpy as jnp
from jax.experimental import pallas as pl

_EXACT = jax.lax.Precision.HIGHEST
_bf16 = jnp.bfloat16

_B = 256
_N_PER = 64
_N = _B * _N_PER
_KNN = 16
_EMB = 64
_NB = 3
_NRBF = 64
_ZPAD = 128
_G = 8
_RHO_PAD = 16

_f32 = jnp.float32


def _knn_select(px, py, pz, need_vec):
    dx = px[:, :, None] - px[:, None, :]
    dy = py[:, :, None] - py[:, None, :]
    dz = pz[:, :, None] - pz[:, None, :]
    d2 = dx * dx + dy * dy + dz * dz
    shape3 = (_G, _N_PER, _N_PER)
    ii = jax.lax.broadcasted_iota(jnp.int32, shape3, 1)
    jj = jax.lax.broadcasted_iota(jnp.int32, shape3, 2)
    d2 = jnp.where(ii == jj, d2 + 1e9, d2)

    s_list, d2_list = [], []
    vx_list, vy_list, vz_list = [], [], []
    d2w = d2
    for _ in range(_KNN):
        mval = jnp.min(d2w, axis=2, keepdims=True)
        cand = jnp.where(d2w == mval, jj, _N_PER)
        jstar = jnp.min(cand, axis=2, keepdims=True)
        hit = jj == jstar
        oneh = hit.astype(_f32)
        s_list.append(oneh)
        d2_list.append(jnp.sum(oneh * d2, axis=2))
        if need_vec:
            vx_list.append(-jnp.sum(oneh * dx, axis=2))
            vy_list.append(-jnp.sum(oneh * dy, axis=2))
            vz_list.append(-jnp.sum(oneh * dz, axis=2))
        d2w = jnp.where(hit, 1e30, d2w)

    S = jnp.stack(s_list, axis=1)
    d2sel = jnp.stack(d2_list, axis=1)
    d_edge = jnp.sqrt(d2sel + 1e-9)
    if need_vec:
        vec = (jnp.stack(vx_list, axis=1),
               jnp.stack(vy_list, axis=1),
               jnp.stack(vz_list, axis=1))
    else:
        vec = None
    return S, d_edge, vec


def _rbf_edges(d_edge):
    cent = (jax.lax.broadcasted_iota(jnp.int32, (1, 1, 1, _NRBF), 3)
            .astype(_f32) * (8.0 / 63.0))
    diff = d_edge[..., None] - cent
    e4 = jnp.exp(-(diff * diff) * 2.0)
    return e4.reshape(_G * _KNN * _N_PER, _NRBF)


def _gather(S, h2):
    sflat = S.reshape(_G, _KNN * _N_PER, _N_PER)
    outs = []
    for g in range(_G):
        hg = h2[g * _N_PER:(g + 1) * _N_PER, :]
        outs.append(jnp.dot(sflat[g], hg, preferred_element_type=_f32,
                            precision=_EXACT))
    return jnp.concatenate(outs, axis=0)


def _segsum_dst(m):
    return (m.reshape(_G, _KNN, _N_PER, _EMB)
             .sum(axis=1)
             .reshape(_G * _N_PER, _EMB))


def _fused_kernel(cell_ref, x_ref, z_ref, emb_ref, enc_cat_ref, enc_upd_ref,
                  wen_ref, din_ref, dec_cat_ref, dec_upd_ref, fw_ref,
                  stress_ref, xp_ref, traj_ref, rho_ref):
    xr = x_ref[:]
    cl = cell_ref[:]

    xrb = xr.astype(_bf16).astype(_f32)
    clb = cl.astype(_bf16).astype(_f32)
    pos = []
    for c in range(3):
        acc = (xrb[:, 0, :] * clb[:, 0:1, c]
               + xrb[:, 1, :] * clb[:, 1:2, c]
               + xrb[:, 2, :] * clb[:, 2:3, c])
        pos.append(acc)

    S1, d1, _ = _knn_select(pos[0], pos[1], pos[2], need_vec=False)
    e1 = _rbf_edges(d1)
    ew1 = jnp.dot(e1.astype(_bf16), enc_cat_ref[:],
                  preferred_element_type=_f32)

    z3 = z_ref[:]
    zq = jax.lax.broadcasted_iota(jnp.int32, (_G, _N_PER, _ZPAD), 2)
    oh = (z3[:, :, None] == zq).astype(_f32)
    h2 = jnp.dot(oh.reshape(_G * _N_PER, _ZPAD), emb_ref[:],
                 preferred_element_type=_f32, precision=_EXACT)

    for b in range(_NB):
        hs = _gather(S1, h2)
        m = hs * ew1[:, b * _EMB:(b + 1) * _EMB]
        agg = _segsum_dst(m)
        upd = jnp.dot(agg.astype(_bf16), enc_upd_ref[b],
                      preferred_element_type=_f32)
        h2 = h2 + jnp.tanh(upd)

    hsum = h2.reshape(_G, _N_PER, _EMB).sum(axis=1)
    hm = jnp.dot(hsum.astype(_bf16), wen_ref[:],
                 preferred_element_type=_f32)

    part_h = jnp.dot(h2.astype(_bf16), din_ref[0:_EMB, :],
                     preferred_element_type=_f32)
    part_g = jnp.dot(hm.astype(_bf16), din_ref[_EMB:2 * _EMB, :],
                     preferred_element_type=_f32)
    hd2 = jnp.tanh(part_h.reshape(_G, _N_PER, _EMB) + part_g[:, None, :])
    hd2 = hd2.reshape(_G * _N_PER, _EMB)

    px2, py2, pz2 = xr[:, 0, :], xr[:, 1, :], xr[:, 2, :]
    S2, d2e, vec2 = _knn_select(px2, py2, pz2, need_vec=True)
    vx2, vy2, vz2 = vec2
    e2 = _rbf_edges(d2e)
    ew2 = jnp.dot(e2.astype(_bf16), dec_cat_ref[:],
                  preferred_element_type=_f32)
    w_e = jnp.exp(-d2e)
    inv_d = 1.0 / d2e

    xpc = [px2, py2, pz2]
    for b in range(_NB):
        hs = _gather(S2, hd2)
        m = hs * ew2[:, b * _EMB:(b + 1) * _EMB]
        agg = _segsum_dst(m)
        upd = jnp.dot(agg.astype(_bf16), dec_upd_ref[b],
                      preferred_element_type=_f32)
        hd2 = hd2 + jnp.tanh(upd)

        hd3 = hd2.astype(_bf16).astype(_f32).reshape(_G, _N_PER, _EMB)
        fwb = fw_ref[b:b + 1, :]
        hf = jnp.sum(hd3 * fwb[:, None, :], axis=2)
        sf = jnp.sum(S2 * hf[:, None, None, :], axis=3)
        su = sf * w_e * inv_d
        for c, v in enumerate((vx2, vy2, vz2)):
            f = jnp.sum(v * su, axis=1)
            xpc[c] = xpc[c] + 0.01 * f
            traj_ref[:, b, c, :] = xpc[c]

    for c in range(3):
        xp_ref[:, c, :] = xpc[c]

    wsum = jnp.sum(w_e, axis=1)
    hd3b = hd2.astype(_bf16).astype(_f32).reshape(_G, _N_PER, _EMB)
    rvec = jnp.sum(hd3b * wsum[:, :, None], axis=1)
    rho_ref[:] = jnp.dot(rvec, stress_ref[:], preferred_element_type=_f32,
                         precision=_EXACT)


def kernel(cell, x, z, num_atoms, enc_emb, enc_edge_w, enc_upd_w, W_energy,
           dec_in_w, dec_edge_w, dec_upd_w, dec_force_w, dec_stress_w):
    del num_atoms

    x_r = x.reshape(_B, _N_PER, 3).transpose(0, 2, 1)
    z_r = z.reshape(_B, _N_PER).astype(jnp.int32)
    emb_pad = jnp.zeros((_ZPAD, _EMB), _f32).at[:enc_emb.shape[0]].set(enc_emb)
    enc_cat = jnp.concatenate([enc_edge_w[i] for i in range(_NB)],
                              axis=1).astype(_bf16)
    dec_cat = jnp.concatenate([dec_edge_w[i] for i in range(_NB)],
                              axis=1).astype(_bf16)
    enc_upd_b = enc_upd_w.astype(_bf16)
    dec_upd_b = dec_upd_w.astype(_bf16)
    wen_b = W_energy.astype(_bf16)
    din_b = dec_in_w.astype(_bf16)
    fw = jnp.zeros((8, _EMB), _f32).at[:_NB].set(
        dec_force_w[:, :, 0].astype(_bf16).astype(_f32))
    stress_pad = jnp.zeros((_EMB, _RHO_PAD), _f32).at[:, :9].set(
        dec_stress_w.astype(_bf16).astype(_f32))

    grid = (_B // _G,)

    xp_r, traj_r, rho_pad = pl.pallas_call(
        _fused_kernel,
        grid=grid,
        in_specs=[
            pl.BlockSpec((_G, 3, 3), lambda i: (i, 0, 0)),
            pl.BlockSpec((_G, 3, _N_PER), lambda i: (i, 0, 0)),
            pl.BlockSpec((_G, _N_PER), lambda i: (i, 0)),
            pl.BlockSpec((_ZPAD, _EMB), lambda i: (0, 0)),
            pl.BlockSpec((_EMB, _NB * _EMB), lambda i: (0, 0)),
            pl.BlockSpec((_NB, _EMB, _EMB), lambda i: (0, 0, 0)),
            pl.BlockSpec((_EMB, _EMB), lambda i: (0, 0)),
            pl.BlockSpec((2 * _EMB, _EMB), lambda i: (0, 0)),
            pl.BlockSpec((_EMB, _NB * _EMB), lambda i: (0, 0)),
            pl.BlockSpec((_NB, _EMB, _EMB), lambda i: (0, 0, 0)),
            pl.BlockSpec((8, _EMB), lambda i: (0, 0)),
            pl.BlockSpec((_EMB, _RHO_PAD), lambda i: (0, 0)),
        ],
        out_specs=[
            pl.BlockSpec((_G, 3, _N_PER), lambda i: (i, 0, 0)),
            pl.BlockSpec((_G, _NB, 3, _N_PER), lambda i: (i, 0, 0, 0)),
            pl.BlockSpec((_G, _RHO_PAD), lambda i: (i, 0)),
        ],
        out_shape=[
            jax.ShapeDtypeStruct((_B, 3, _N_PER), _f32),
            jax.ShapeDtypeStruct((_B, _NB, 3, _N_PER), _f32),
            jax.ShapeDtypeStruct((_B, _RHO_PAD), _f32),
        ],
        interpret=False,
    )(cell, x_r, z_r, emb_pad, enc_cat, enc_upd_b, wen_b, din_b,
      dec_cat, dec_upd_b, fw, stress_pad)

    xp = xp_r.transpose(0, 2, 1).reshape(_N, 3)
    x_traj = traj_r.transpose(1, 0, 3, 2).reshape(_NB, _N, 3)
    rho = rho_pad[:, :9].reshape(_B, 3, 3)
    return xp, x_traj, rho
